# Initial kernel scaffold; baseline (speedup 1.0000x reference)
import jax
import jax.numpy as jnp
from jax.experimental import pallas as pl


def kernel(x, perms):
    raise NotImplementedError("write your pallas kernel here")


# speedup vs baseline: 6.0209x; 6.0209x over previous
import jax
import jax.numpy as jnp
from jax.experimental import pallas as pl
from jax.experimental.pallas import tpu as pltpu

_S = 3
_B = 8192
_D = 1024
_G = 64
_d = 16
_SG = _S * _G
_NS_ITERS = 14

_RB1 = 512
_R1 = _B // _RB1


def _moments_kernel(x_ref, gram_ref, cs_ref):
    r = pl.program_id(1)
    xb = x_ref[...]
    xh = xb.astype(jnp.bfloat16)
    g = jax.lax.dot_general(
        xh, xh, (((0,), (0,)), ((), ())),
        preferred_element_type=jnp.float32)
    cs = jnp.sum(xb, axis=0, keepdims=True)

    @pl.when(r == 0)
    def _():
        gram_ref[...] = g[None]
        cs_ref[...] = cs[None]

    @pl.when(r != 0)
    def _():
        gram_ref[...] += g[None]
        cs_ref[...] += cs[None]


def _moments(x):
    return pl.pallas_call(
        _moments_kernel,
        grid=(_S, _R1),
        in_specs=[pl.BlockSpec(
            (_RB1, _D), lambda s, r: (s * _R1 + r, 0))],
        out_specs=[
            pl.BlockSpec((1, _D, _D), lambda s, r: (s, 0, 0)),
            pl.BlockSpec((1, 1, _D), lambda s, r: (s, 0, 0)),
        ],
        out_shape=[
            jax.ShapeDtypeStruct((_S, _D, _D), jnp.float32),
            jax.ShapeDtypeStruct((_S, 1, _D), jnp.float32),
        ],
        compiler_params=pltpu.CompilerParams(
            dimension_semantics=("arbitrary", "arbitrary"),
            vmem_limit_bytes=48 * 1024 * 1024,
        ),
    )(x)


def _bmm(p, q):
    return jnp.sum(p[:, :, None, :] * q[None, :, :, :], axis=1)


def _whiten_kernel(bl_ref, mu_ref, w_ref):
    m = mu_ref[...] * (1.0 / _B)
    cov = bl_ref[...] * (1.0 / _B) - m[:, None, :] * m[None, :, :]
    nrm = jnp.sqrt(jnp.sum(cov * cov, axis=(0, 1), keepdims=True))
    a = cov * (1.0 / nrm)
    eye = (jax.lax.broadcasted_iota(jnp.int32, (_d, _d, 1), 0) ==
           jax.lax.broadcasted_iota(jnp.int32, (_d, _d, 1), 1)
           ).astype(jnp.float32)
    y = a
    z = jnp.broadcast_to(eye, a.shape)
    for _ in range(_NS_ITERS):
        t = 1.5 * eye - 0.5 * _bmm(z, y)
        y = _bmm(y, t)
        z = _bmm(t, z)
    w_ref[...] = z * jax.lax.rsqrt(nrm)


def _whiten(blocks, mu_g):
    return pl.pallas_call(
        _whiten_kernel,
        out_shape=jax.ShapeDtypeStruct((_d, _d, _SG), jnp.float32),
    )(blocks, mu_g)


_RB3 = 1024
_T3 = (_S * _B) // _RB3
_VPB = _B // _RB3


def _apply_kernel(x_ref, m_ref, mu_ref, y_ref):
    xc = x_ref[...] - mu_ref[0]
    y_ref[...] = jax.lax.dot_general(
        xc.astype(jnp.bfloat16), m_ref[0],
        (((1,), (0,)), ((), ())),
        preferred_element_type=jnp.float32)


def _apply(x, m_bf, mu):
    return pl.pallas_call(
        _apply_kernel,
        grid=(_T3,),
        in_specs=[
            pl.BlockSpec((_RB3, _D), lambda t: (t, 0)),
            pl.BlockSpec((1, _D, _D), lambda t: (t // _VPB, 0, 0)),
            pl.BlockSpec((1, 1, _D), lambda t: (t // _VPB, 0, 0)),
        ],
        out_specs=pl.BlockSpec((_RB3, _D), lambda t: (t, 0)),
        out_shape=jax.ShapeDtypeStruct((_S * _B, _D), jnp.float32),
        compiler_params=pltpu.CompilerParams(
            dimension_semantics=("arbitrary",),
            vmem_limit_bytes=48 * 1024 * 1024,
        ),
    )(x, m_bf, mu)


def kernel(x, perms):
    perms = perms.astype(jnp.int32)
    gram, cs = _moments(x)
    cs = cs.reshape(_S, _D)

    pg = perms.reshape(_S, _G, _d)
    g1 = jnp.take_along_axis(gram, perms[:, :, None], axis=1)
    g1 = g1.reshape(_S, _G, _d, _D)
    blocks = jnp.take_along_axis(g1, pg[:, :, None, :], axis=3)
    bk2 = blocks.transpose(2, 3, 0, 1).reshape(_d, _d, _SG)
    csg = jnp.take_along_axis(cs, perms, axis=1).reshape(_S, _G, _d)
    mk2 = csg.transpose(2, 0, 1).reshape(_d, _SG)

    w = _whiten(bk2, mk2)

    wt = w.reshape(_d, _d, _S, _G).transpose(2, 3, 0, 1)
    wbd = (wt[:, :, :, None, :] *
           jnp.eye(_G, dtype=w.dtype)[None, :, None, :, None]
           ).reshape(_S, _D, _D)
    inv = jnp.zeros((_S, _D), jnp.int32).at[
        jnp.arange(_S)[:, None], perms].set(
        jnp.broadcast_to(jnp.arange(_D, dtype=jnp.int32)[None], (_S, _D)))
    m_mat = jnp.take_along_axis(wbd, inv[:, :, None], axis=1)
    m_mat = jnp.take_along_axis(m_mat, inv[:, None, :], axis=2)
    m_bf = m_mat.astype(jnp.bfloat16)
    mu = (cs * (1.0 / _B)).reshape(_S, 1, _D)

    return _apply(x, m_bf, mu)


# speedup vs baseline: 13.5231x; 2.2460x over previous
import jax
import jax.numpy as jnp
from jax.experimental import pallas as pl
from jax.experimental.pallas import tpu as pltpu

_S = 3
_B = 8192
_D = 1024
_G = 64
_d = 16
_SG = _S * _G
_NS_ITERS = 14

_RB1 = 512
_R1 = _B // _RB1


def _moments_kernel(x_ref, gram_ref, cs_ref):
    r = pl.program_id(1)
    xb = x_ref[...]
    xh = xb.astype(jnp.bfloat16)
    g = jax.lax.dot_general(
        xh, xh, (((0,), (0,)), ((), ())),
        preferred_element_type=jnp.float32)
    cs = jnp.sum(xb, axis=0, keepdims=True)

    @pl.when(r == 0)
    def _():
        gram_ref[...] = g[None]
        cs_ref[...] = cs[None]

    @pl.when(r != 0)
    def _():
        gram_ref[...] += g[None]
        cs_ref[...] += cs[None]


def _moments(x):
    return pl.pallas_call(
        _moments_kernel,
        grid=(_S, _R1),
        in_specs=[pl.BlockSpec(
            (_RB1, _D), lambda s, r: (s * _R1 + r, 0))],
        out_specs=[
            pl.BlockSpec((1, _D, _D), lambda s, r: (s, 0, 0)),
            pl.BlockSpec((1, 1, _D), lambda s, r: (s, 0, 0)),
        ],
        out_shape=[
            jax.ShapeDtypeStruct((_S, _D, _D), jnp.float32),
            jax.ShapeDtypeStruct((_S, 1, _D), jnp.float32),
        ],
        compiler_params=pltpu.CompilerParams(
            dimension_semantics=("arbitrary", "arbitrary"),
            vmem_limit_bytes=48 * 1024 * 1024,
        ),
    )(x)


def _middle_kernel(gram_ref, csp_ref, perm_ref, m_ref):
    f32 = jnp.float32
    bf16 = jnp.bfloat16

    iota_r = jax.lax.broadcasted_iota(jnp.int32, (_D, _D), 0)
    iota_c = jax.lax.broadcasted_iota(jnp.int32, (_D, _D), 1)
    p_row = jnp.broadcast_to(perm_ref[0], (_D, _D))
    e_bf = jnp.where(iota_r == p_row, f32(1), f32(0)).astype(bf16)

    g_bf = gram_ref[0].astype(bf16)
    t1 = jax.lax.dot_general(e_bf, g_bf, (((0,), (0,)), ((), ())),
                             preferred_element_type=f32)
    gp = jax.lax.dot_general(t1.astype(bf16), e_bf, (((1,), (0,)), ((), ())),
                             preferred_element_type=f32)

    mu_p = csp_ref[0] * (1.0 / _B)
    mumu = jax.lax.dot_general(mu_p, mu_p, (((0,), (0,)), ((), ())),
                               preferred_element_type=f32,
                               precision=jax.lax.Precision.HIGHEST)
    covf = gp * (1.0 / _B) - mumu

    a2 = jnp.concatenate(
        [covf[g * _d:(g + 1) * _d, g * _d:(g + 1) * _d] for g in range(_G)],
        axis=1)

    blk_mask = (iota_r >> 4) == (iota_c >> 4)
    s_bf = jnp.where(blk_mask, f32(1), f32(0)).astype(bf16)
    rs = jnp.sum(a2 * a2, axis=0, keepdims=True)
    nrm = jax.lax.dot_general(rs.astype(bf16), s_bf, (((1,), (0,)), ((), ())),
                              preferred_element_type=f32) * 1.02
    inv_nrm = 1.0 / nrm

    eye2 = (jax.lax.broadcasted_iota(jnp.int32, (_d, _D), 0) ==
            (jax.lax.broadcasted_iota(jnp.int32, (_d, _D), 1) & (_d - 1))
            ).astype(f32)
    def _bd(q_bf):
        rep = pltpu.repeat(q_bf, _G, axis=0)
        return rep * s_bf

    def _bmm(p, q_bf):
        return jax.lax.dot_general(p.astype(bf16), _bd(q_bf),
                                   (((1,), (0,)), ((), ())),
                                   preferred_element_type=f32)

    y = a2 * inv_nrm
    z = eye2
    for _ in range(_NS_ITERS):
        y_bf = y.astype(bf16)
        t = 1.5 * eye2 - 0.5 * _bmm(z, y_bf)
        y = _bmm(t.astype(bf16), y_bf)
        z = _bmm(t.astype(bf16), z.astype(bf16))
    w2 = z * jax.lax.rsqrt(nrm)

    t2 = jax.lax.dot_general(_bd(w2.astype(bf16)), e_bf, (((1,), (1,)), ((), ())),
                             preferred_element_type=f32)
    m = jax.lax.dot_general(e_bf, t2.astype(bf16), (((1,), (0,)), ((), ())),
                            preferred_element_type=f32)
    m_ref[...] = m.astype(bf16)[None]


def _middle(gram, cs_pg, perms3):
    return pl.pallas_call(
        _middle_kernel,
        grid=(_S,),
        in_specs=[
            pl.BlockSpec((1, _D, _D), lambda s: (s, 0, 0)),
            pl.BlockSpec((1, 1, _D), lambda s: (s, 0, 0)),
            pl.BlockSpec((1, 1, _D), lambda s: (s, 0, 0)),
        ],
        out_specs=pl.BlockSpec((1, _D, _D), lambda s: (s, 0, 0)),
        out_shape=jax.ShapeDtypeStruct((_S, _D, _D), jnp.bfloat16),
        compiler_params=pltpu.CompilerParams(
            dimension_semantics=("arbitrary",),
            vmem_limit_bytes=48 * 1024 * 1024,
        ),
    )(gram, cs_pg, perms3)


_RB3 = 1024
_T3 = (_S * _B) // _RB3
_VPB = _B // _RB3


def _apply_kernel(x_ref, m_ref, mu_ref, y_ref):
    xc = x_ref[...] - mu_ref[0]
    y_ref[...] = jax.lax.dot_general(
        xc.astype(jnp.bfloat16), m_ref[0],
        (((1,), (0,)), ((), ())),
        preferred_element_type=jnp.float32)


def _apply(x, m_bf, mu):
    return pl.pallas_call(
        _apply_kernel,
        grid=(_T3,),
        in_specs=[
            pl.BlockSpec((_RB3, _D), lambda t: (t, 0)),
            pl.BlockSpec((1, _D, _D), lambda t: (t // _VPB, 0, 0)),
            pl.BlockSpec((1, 1, _D), lambda t: (t // _VPB, 0, 0)),
        ],
        out_specs=pl.BlockSpec((_RB3, _D), lambda t: (t, 0)),
        out_shape=jax.ShapeDtypeStruct((_S * _B, _D), jnp.float32),
        compiler_params=pltpu.CompilerParams(
            dimension_semantics=("arbitrary",),
            vmem_limit_bytes=48 * 1024 * 1024,
        ),
    )(x, m_bf, mu)


def kernel(x, perms):
    perms = perms.astype(jnp.int32)
    gram, cs = _moments(x)
    cs_pg = jnp.take_along_axis(cs[:, 0, :], perms, axis=1)[:, None, :]
    m_bf = _middle(gram, cs_pg, perms[:, None, :])
    mu = cs * (1.0 / _B)
    return _apply(x, m_bf, mu)


# speedup vs baseline: 15.1674x; 1.1216x over previous
import jax
import jax.numpy as jnp
from jax.experimental import pallas as pl
from jax.experimental.pallas import tpu as pltpu

_S = 3
_B = 8192
_D = 1024
_G = 64
_d = 16
_SG = _S * _G
_NS_ITERS = 14

_RB1 = 2048
_R1 = _B // _RB1


def _moments_kernel(x_ref, gram_ref, cs_ref):
    r = pl.program_id(1)
    xb = x_ref[...]
    xh = xb.astype(jnp.bfloat16)
    g = jax.lax.dot_general(
        xh, xh, (((0,), (0,)), ((), ())),
        preferred_element_type=jnp.float32)
    cs = jnp.sum(xb, axis=0, keepdims=True)

    @pl.when(r == 0)
    def _():
        gram_ref[...] = g[None]
        cs_ref[...] = cs[None]

    @pl.when(r != 0)
    def _():
        gram_ref[...] += g[None]
        cs_ref[...] += cs[None]


def _moments(x):
    return pl.pallas_call(
        _moments_kernel,
        grid=(_S, _R1),
        in_specs=[pl.BlockSpec(
            (_RB1, _D), lambda s, r: (s * _R1 + r, 0))],
        out_specs=[
            pl.BlockSpec((1, _D, _D), lambda s, r: (s, 0, 0)),
            pl.BlockSpec((1, 1, _D), lambda s, r: (s, 0, 0)),
        ],
        out_shape=[
            jax.ShapeDtypeStruct((_S, _D, _D), jnp.float32),
            jax.ShapeDtypeStruct((_S, 1, _D), jnp.float32),
        ],
        compiler_params=pltpu.CompilerParams(
            dimension_semantics=("arbitrary", "arbitrary"),
            vmem_limit_bytes=48 * 1024 * 1024,
        ),
    )(x)


def _middle_kernel(gram_ref, csp_ref, perm_ref, m_ref):
    f32 = jnp.float32
    bf16 = jnp.bfloat16

    iota_r = jax.lax.broadcasted_iota(jnp.int32, (_D, _D), 0)
    iota_c = jax.lax.broadcasted_iota(jnp.int32, (_D, _D), 1)
    p_row = jnp.broadcast_to(perm_ref[0], (_D, _D))
    e_bf = jnp.where(iota_r == p_row, f32(1), f32(0)).astype(bf16)

    g_bf = gram_ref[0].astype(bf16)
    t1 = jax.lax.dot_general(e_bf, g_bf, (((0,), (0,)), ((), ())),
                             preferred_element_type=f32)
    gp = jax.lax.dot_general(t1.astype(bf16), e_bf, (((1,), (0,)), ((), ())),
                             preferred_element_type=f32)

    mu_p = csp_ref[0] * (1.0 / _B)
    mumu = jax.lax.dot_general(mu_p, mu_p, (((0,), (0,)), ((), ())),
                               preferred_element_type=f32,
                               precision=jax.lax.Precision.HIGHEST)
    covf = gp * (1.0 / _B) - mumu

    a2 = jnp.concatenate(
        [covf[g * _d:(g + 1) * _d, g * _d:(g + 1) * _d] for g in range(_G)],
        axis=1)

    blk_mask = (iota_r >> 4) == (iota_c >> 4)
    s_bf = jnp.where(blk_mask, f32(1), f32(0)).astype(bf16)
    rs = jnp.sum(a2 * a2, axis=0, keepdims=True)
    nrm = jax.lax.dot_general(rs.astype(bf16), s_bf, (((1,), (0,)), ((), ())),
                              preferred_element_type=f32) * 1.02
    inv_nrm = 1.0 / nrm

    eye2 = (jax.lax.broadcasted_iota(jnp.int32, (_d, _D), 0) ==
            (jax.lax.broadcasted_iota(jnp.int32, (_d, _D), 1) & (_d - 1))
            ).astype(f32)
    def _bd(q_bf):
        rep = pltpu.repeat(q_bf, _G, axis=0)
        return rep * s_bf

    def _bmm(p, q_bf):
        return jax.lax.dot_general(p.astype(bf16), _bd(q_bf),
                                   (((1,), (0,)), ((), ())),
                                   preferred_element_type=f32)

    y = a2 * inv_nrm
    z = eye2
    for _ in range(_NS_ITERS):
        y_bf = y.astype(bf16)
        t = 1.5 * eye2 - 0.5 * _bmm(z, y_bf)
        y = _bmm(t.astype(bf16), y_bf)
        z = _bmm(t.astype(bf16), z.astype(bf16))
    w2 = z * jax.lax.rsqrt(nrm)

    t2 = jax.lax.dot_general(_bd(w2.astype(bf16)), e_bf, (((1,), (1,)), ((), ())),
                             preferred_element_type=f32)
    m = jax.lax.dot_general(e_bf, t2.astype(bf16), (((1,), (0,)), ((), ())),
                            preferred_element_type=f32)
    m_ref[...] = m.astype(bf16)[None]


def _middle(gram, cs_pg, perms3):
    return pl.pallas_call(
        _middle_kernel,
        grid=(_S,),
        in_specs=[
            pl.BlockSpec((1, _D, _D), lambda s: (s, 0, 0)),
            pl.BlockSpec((1, 1, _D), lambda s: (s, 0, 0)),
            pl.BlockSpec((1, 1, _D), lambda s: (s, 0, 0)),
        ],
        out_specs=pl.BlockSpec((1, _D, _D), lambda s: (s, 0, 0)),
        out_shape=jax.ShapeDtypeStruct((_S, _D, _D), jnp.bfloat16),
        compiler_params=pltpu.CompilerParams(
            dimension_semantics=("arbitrary",),
            vmem_limit_bytes=48 * 1024 * 1024,
        ),
    )(gram, cs_pg, perms3)


_RB3 = 2048
_T3 = (_S * _B) // _RB3
_VPB = _B // _RB3


def _apply_kernel(x_ref, m_ref, mu_ref, y_ref):
    xc = x_ref[...] - mu_ref[0]
    y_ref[...] = jax.lax.dot_general(
        xc.astype(jnp.bfloat16), m_ref[0],
        (((1,), (0,)), ((), ())),
        preferred_element_type=jnp.float32)


def _apply(x, m_bf, mu):
    return pl.pallas_call(
        _apply_kernel,
        grid=(_T3,),
        in_specs=[
            pl.BlockSpec((_RB3, _D), lambda t: (t, 0)),
            pl.BlockSpec((1, _D, _D), lambda t: (t // _VPB, 0, 0)),
            pl.BlockSpec((1, 1, _D), lambda t: (t // _VPB, 0, 0)),
        ],
        out_specs=pl.BlockSpec((_RB3, _D), lambda t: (t, 0)),
        out_shape=jax.ShapeDtypeStruct((_S * _B, _D), jnp.float32),
        compiler_params=pltpu.CompilerParams(
            dimension_semantics=("arbitrary",),
            vmem_limit_bytes=48 * 1024 * 1024,
        ),
    )(x, m_bf, mu)


def kernel(x, perms):
    perms = perms.astype(jnp.int32)
    gram, cs = _moments(x)
    cs_pg = jnp.take_along_axis(cs[:, 0, :], perms, axis=1)[:, None, :]
    m_bf = _middle(gram, cs_pg, perms[:, None, :])
    mu = cs * (1.0 / _B)
    return _apply(x, m_bf, mu)


# speedup vs baseline: 16.8949x; 1.1139x over previous
import jax
import jax.numpy as jnp
from jax.experimental import pallas as pl
from jax.experimental.pallas import tpu as pltpu

_S = 3
_B = 8192
_D = 1024
_G = 64
_d = 16
_NS_ITERS = 14

_RB1 = 2048
_R1 = _B // _RB1


def _make_m(gram, cs, perm_row):
    f32 = jnp.float32
    bf16 = jnp.bfloat16

    mu = cs * (1.0 / _B)
    mumu = jax.lax.dot_general(mu, mu, (((0,), (0,)), ((), ())),
                               preferred_element_type=f32,
                               precision=jax.lax.Precision.HIGHEST)
    covf = gram * (1.0 / _B) - mumu

    iota_r = jax.lax.broadcasted_iota(jnp.int32, (_D, _D), 0)
    iota_c = jax.lax.broadcasted_iota(jnp.int32, (_D, _D), 1)
    p_row = jnp.broadcast_to(perm_row, (_D, _D))
    e_bf = jnp.where(iota_r == p_row, f32(1), f32(0)).astype(bf16)

    t1 = jax.lax.dot_general(e_bf, covf.astype(bf16), (((0,), (0,)), ((), ())),
                             preferred_element_type=f32)
    cp = jax.lax.dot_general(t1.astype(bf16), e_bf, (((1,), (0,)), ((), ())),
                             preferred_element_type=f32)

    a2 = jnp.concatenate(
        [cp[g * _d:(g + 1) * _d, g * _d:(g + 1) * _d] for g in range(_G)],
        axis=1)

    blk_mask = (iota_r >> 4) == (iota_c >> 4)
    s_bf = jnp.where(blk_mask, f32(1), f32(0)).astype(bf16)
    rs = jnp.sum(a2 * a2, axis=0, keepdims=True)
    nrm = jax.lax.dot_general(rs.astype(bf16), s_bf, (((1,), (0,)), ((), ())),
                              preferred_element_type=f32) * 1.02
    inv_nrm = 1.0 / nrm

    eye2 = (jax.lax.broadcasted_iota(jnp.int32, (_d, _D), 0) ==
            (jax.lax.broadcasted_iota(jnp.int32, (_d, _D), 1) & (_d - 1))
            ).astype(f32)

    def _bd(q_bf):
        rep = pltpu.repeat(q_bf, _G, axis=0)
        return rep * s_bf

    def _bmm(p_bf, q_bf):
        return jax.lax.dot_general(p_bf, _bd(q_bf), (((1,), (0,)), ((), ())),
                                   preferred_element_type=f32)

    y = a2 * inv_nrm
    z = eye2
    for _ in range(_NS_ITERS):
        y_bf = y.astype(bf16)
        t = 1.5 * eye2 - 0.5 * _bmm(z.astype(bf16), y_bf)
        t_bf = t.astype(bf16)
        y = _bmm(t_bf, y_bf)
        z = _bmm(t_bf, z.astype(bf16))
    w2 = z * jax.lax.rsqrt(nrm)

    t2 = jax.lax.dot_general(_bd(w2.astype(bf16)), e_bf,
                             (((1,), (1,)), ((), ())),
                             preferred_element_type=f32)
    m = jax.lax.dot_general(e_bf, t2.astype(bf16), (((1,), (0,)), ((), ())),
                            preferred_element_type=f32)
    return m.astype(bf16)


def _fused_kernel(x_ref, perm_ref, m_ref, cs_ref, gram_ref):
    r = pl.program_id(1)
    xb = x_ref[...]
    xh = xb.astype(jnp.bfloat16)
    g = jax.lax.dot_general(
        xh, xh, (((0,), (0,)), ((), ())),
        preferred_element_type=jnp.float32)
    cs = jnp.sum(xb, axis=0, keepdims=True)

    @pl.when(r == 0)
    def _():
        gram_ref[...] = g
        cs_ref[...] = cs[None]

    @pl.when(r != 0)
    def _():
        gram_ref[...] += g
        cs_ref[...] += cs[None]

    @pl.when(r == _R1 - 1)
    def _():
        m_ref[...] = _make_m(gram_ref[...], cs_ref[0], perm_ref[0])[None]


def _moments_m(x, perms3):
    return pl.pallas_call(
        _fused_kernel,
        grid=(_S, _R1),
        in_specs=[
            pl.BlockSpec((_RB1, _D), lambda s, r: (s * _R1 + r, 0)),
            pl.BlockSpec((1, 1, _D), lambda s, r: (s, 0, 0)),
        ],
        out_specs=[
            pl.BlockSpec((1, _D, _D), lambda s, r: (s, 0, 0)),
            pl.BlockSpec((1, 1, _D), lambda s, r: (s, 0, 0)),
        ],
        out_shape=[
            jax.ShapeDtypeStruct((_S, _D, _D), jnp.bfloat16),
            jax.ShapeDtypeStruct((_S, 1, _D), jnp.float32),
        ],
        scratch_shapes=[pltpu.VMEM((_D, _D), jnp.float32)],
        compiler_params=pltpu.CompilerParams(
            dimension_semantics=("arbitrary", "arbitrary"),
            vmem_limit_bytes=56 * 1024 * 1024,
        ),
    )(x, perms3)


_RB3 = 2048
_T3 = (_S * _B) // _RB3
_VPB = _B // _RB3


def _apply_kernel(x_ref, m_ref, cs_ref, y_ref):
    xc = x_ref[...] - cs_ref[0] * (1.0 / _B)
    y_ref[...] = jax.lax.dot_general(
        xc.astype(jnp.bfloat16), m_ref[0],
        (((1,), (0,)), ((), ())),
        preferred_element_type=jnp.float32)


def _apply(x, m_bf, cs):
    return pl.pallas_call(
        _apply_kernel,
        grid=(_T3,),
        in_specs=[
            pl.BlockSpec((_RB3, _D), lambda t: (t, 0)),
            pl.BlockSpec((1, _D, _D), lambda t: (t // _VPB, 0, 0)),
            pl.BlockSpec((1, 1, _D), lambda t: (t // _VPB, 0, 0)),
        ],
        out_specs=pl.BlockSpec((_RB3, _D), lambda t: (t, 0)),
        out_shape=jax.ShapeDtypeStruct((_S * _B, _D), jnp.float32),
        compiler_params=pltpu.CompilerParams(
            dimension_semantics=("arbitrary",),
            vmem_limit_bytes=48 * 1024 * 1024,
        ),
    )(x, m_bf, cs)


def kernel(x, perms):
    m_bf, cs = _moments_m(x, perms.astype(jnp.int32)[:, None, :])
    return _apply(x, m_bf, cs)


# speedup vs baseline: 17.7555x; 1.0509x over previous
import jax
import jax.numpy as jnp
from jax.experimental import pallas as pl
from jax.experimental.pallas import tpu as pltpu

_S = 3
_B = 8192
_D = 1024
_G = 64
_d = 16
_NS_ITERS = 10

_RB1 = 2048
_R1 = _B // _RB1


def _make_m(gram, cs, perm_row):
    f32 = jnp.float32
    bf16 = jnp.bfloat16

    mu = cs * (1.0 / _B)
    mumu = jax.lax.dot_general(mu, mu, (((0,), (0,)), ((), ())),
                               preferred_element_type=f32,
                               precision=jax.lax.Precision.HIGHEST)
    covf = gram * (1.0 / _B) - mumu

    iota_r = jax.lax.broadcasted_iota(jnp.int32, (_D, _D), 0)
    iota_c = jax.lax.broadcasted_iota(jnp.int32, (_D, _D), 1)
    p_row = jnp.broadcast_to(perm_row, (_D, _D))
    e_bf = jnp.where(iota_r == p_row, f32(1), f32(0)).astype(bf16)

    t1 = jax.lax.dot_general(e_bf, covf.astype(bf16), (((0,), (0,)), ((), ())),
                             preferred_element_type=f32)
    cp = jax.lax.dot_general(t1.astype(bf16), e_bf, (((1,), (0,)), ((), ())),
                             preferred_element_type=f32)

    a2 = jnp.concatenate(
        [cp[g * _d:(g + 1) * _d, g * _d:(g + 1) * _d] for g in range(_G)],
        axis=1)

    blk_mask = (iota_r >> 4) == (iota_c >> 4)
    s_bf = jnp.where(blk_mask, f32(1), f32(0)).astype(bf16)
    rs = jnp.sum(a2 * a2, axis=0, keepdims=True)
    nrm = jax.lax.dot_general(rs.astype(bf16), s_bf, (((1,), (0,)), ((), ())),
                              preferred_element_type=f32) * 1.02
    inv_nrm = 1.0 / nrm

    eye2 = (jax.lax.broadcasted_iota(jnp.int32, (_d, _D), 0) ==
            (jax.lax.broadcasted_iota(jnp.int32, (_d, _D), 1) & (_d - 1))
            ).astype(f32)

    def _bd(q_bf):
        rep = pltpu.repeat(q_bf, _G, axis=0)
        return rep * s_bf

    def _bmm(p_bf, q_bf):
        return jax.lax.dot_general(p_bf, _bd(q_bf), (((1,), (0,)), ((), ())),
                                   preferred_element_type=f32)

    y = a2 * inv_nrm
    z = eye2
    for _ in range(_NS_ITERS):
        y_bf = y.astype(bf16)
        t = 1.5 * eye2 - 0.5 * _bmm(z.astype(bf16), y_bf)
        t_bf = t.astype(bf16)
        y = _bmm(t_bf, y_bf)
        z = _bmm(t_bf, z.astype(bf16))
    w2 = z * jax.lax.rsqrt(nrm)

    t2 = jax.lax.dot_general(_bd(w2.astype(bf16)), e_bf,
                             (((1,), (1,)), ((), ())),
                             preferred_element_type=f32)
    m = jax.lax.dot_general(e_bf, t2.astype(bf16), (((1,), (0,)), ((), ())),
                            preferred_element_type=f32)
    return m.astype(bf16)


def _fused_kernel(x_ref, perm_ref, m_ref, cs_ref, gram_ref):
    r = pl.program_id(1)
    xb = x_ref[...]
    xh = xb.astype(jnp.bfloat16)
    g = jax.lax.dot_general(
        xh, xh, (((0,), (0,)), ((), ())),
        preferred_element_type=jnp.float32)
    cs = jnp.sum(xb, axis=0, keepdims=True)

    @pl.when(r == 0)
    def _():
        gram_ref[...] = g
        cs_ref[...] = cs[None]

    @pl.when(r != 0)
    def _():
        gram_ref[...] += g
        cs_ref[...] += cs[None]

    @pl.when(r == _R1 - 1)
    def _():
        m_ref[...] = _make_m(gram_ref[...], cs_ref[0], perm_ref[0])[None]


def _moments_m(x, perms3):
    return pl.pallas_call(
        _fused_kernel,
        grid=(_S, _R1),
        in_specs=[
            pl.BlockSpec((_RB1, _D), lambda s, r: (s * _R1 + r, 0)),
            pl.BlockSpec((1, 1, _D), lambda s, r: (s, 0, 0)),
        ],
        out_specs=[
            pl.BlockSpec((1, _D, _D), lambda s, r: (s, 0, 0)),
            pl.BlockSpec((1, 1, _D), lambda s, r: (s, 0, 0)),
        ],
        out_shape=[
            jax.ShapeDtypeStruct((_S, _D, _D), jnp.bfloat16),
            jax.ShapeDtypeStruct((_S, 1, _D), jnp.float32),
        ],
        scratch_shapes=[pltpu.VMEM((_D, _D), jnp.float32)],
        compiler_params=pltpu.CompilerParams(
            dimension_semantics=("arbitrary", "arbitrary"),
            vmem_limit_bytes=56 * 1024 * 1024,
        ),
    )(x, perms3)


_RB3 = 2048
_T3 = (_S * _B) // _RB3
_VPB = _B // _RB3


def _apply_kernel(x_ref, m_ref, cs_ref, y_ref):
    xc = x_ref[...] - cs_ref[0] * (1.0 / _B)
    y_ref[...] = jax.lax.dot_general(
        xc.astype(jnp.bfloat16), m_ref[0],
        (((1,), (0,)), ((), ())),
        preferred_element_type=jnp.float32)


def _apply(x, m_bf, cs):
    return pl.pallas_call(
        _apply_kernel,
        grid=(_T3,),
        in_specs=[
            pl.BlockSpec((_RB3, _D), lambda t: (t, 0)),
            pl.BlockSpec((1, _D, _D), lambda t: (t // _VPB, 0, 0)),
            pl.BlockSpec((1, 1, _D), lambda t: (t // _VPB, 0, 0)),
        ],
        out_specs=pl.BlockSpec((_RB3, _D), lambda t: (t, 0)),
        out_shape=jax.ShapeDtypeStruct((_S * _B, _D), jnp.float32),
        compiler_params=pltpu.CompilerParams(
            dimension_semantics=("arbitrary",),
            vmem_limit_bytes=48 * 1024 * 1024,
        ),
    )(x, m_bf, cs)


def kernel(x, perms):
    m_bf, cs = _moments_m(x, perms.astype(jnp.int32)[:, None, :])
    return _apply(x, m_bf, cs)


# speedup vs baseline: 18.1926x; 1.0246x over previous
import jax
import jax.numpy as jnp
from jax.experimental import pallas as pl
from jax.experimental.pallas import tpu as pltpu

_S = 3
_B = 8192
_D = 1024
_G = 64
_d = 16
_NS_ITERS = 8

_RB1 = 2048
_R1 = _B // _RB1


def _make_m(gram, cs, perm_row):
    f32 = jnp.float32
    bf16 = jnp.bfloat16

    mu = cs * (1.0 / _B)
    mumu = jax.lax.dot_general(mu, mu, (((0,), (0,)), ((), ())),
                               preferred_element_type=f32,
                               precision=jax.lax.Precision.HIGHEST)
    covf = gram * (1.0 / _B) - mumu

    iota_r = jax.lax.broadcasted_iota(jnp.int32, (_D, _D), 0)
    iota_c = jax.lax.broadcasted_iota(jnp.int32, (_D, _D), 1)
    p_row = jnp.broadcast_to(perm_row, (_D, _D))
    e_bf = jnp.where(iota_r == p_row, f32(1), f32(0)).astype(bf16)

    t1 = jax.lax.dot_general(e_bf, covf.astype(bf16), (((0,), (0,)), ((), ())),
                             preferred_element_type=f32)
    cp = jax.lax.dot_general(t1.astype(bf16), e_bf, (((1,), (0,)), ((), ())),
                             preferred_element_type=f32)

    a2 = jnp.concatenate(
        [cp[g * _d:(g + 1) * _d, g * _d:(g + 1) * _d] for g in range(_G)],
        axis=1)

    blk_mask = (iota_r >> 4) == (iota_c >> 4)
    s_bf = jnp.where(blk_mask, f32(1), f32(0)).astype(bf16)
    rs = jnp.sum(a2 * a2, axis=0, keepdims=True)
    nrm = jax.lax.dot_general(rs.astype(bf16), s_bf, (((1,), (0,)), ((), ())),
                              preferred_element_type=f32) * 1.02
    inv_nrm = 1.0 / nrm

    eye2 = (jax.lax.broadcasted_iota(jnp.int32, (_d, _D), 0) ==
            (jax.lax.broadcasted_iota(jnp.int32, (_d, _D), 1) & (_d - 1))
            ).astype(f32)

    def _bd(q_bf):
        rep = pltpu.repeat(q_bf, _G, axis=0)
        return rep * s_bf

    def _bmm(p_bf, q_bf):
        return jax.lax.dot_general(p_bf, _bd(q_bf), (((1,), (0,)), ((), ())),
                                   preferred_element_type=f32)

    y = a2 * inv_nrm
    z = eye2
    for _ in range(_NS_ITERS):
        y_bf = y.astype(bf16)
        t = 1.5 * eye2 - 0.5 * _bmm(z.astype(bf16), y_bf)
        t_bf = t.astype(bf16)
        y = _bmm(t_bf, y_bf)
        z = _bmm(t_bf, z.astype(bf16))
    w2 = z * jax.lax.rsqrt(nrm)

    t2 = jax.lax.dot_general(_bd(w2.astype(bf16)), e_bf,
                             (((1,), (1,)), ((), ())),
                             preferred_element_type=f32)
    m = jax.lax.dot_general(e_bf, t2.astype(bf16), (((1,), (0,)), ((), ())),
                            preferred_element_type=f32)
    return m.astype(bf16)


def _fused_kernel(x_ref, perm_ref, m_ref, cs_ref, gram_ref):
    r = pl.program_id(1)
    xb = x_ref[...]
    xh = xb.astype(jnp.bfloat16)
    g = jax.lax.dot_general(
        xh, xh, (((0,), (0,)), ((), ())),
        preferred_element_type=jnp.float32)
    cs = jnp.sum(xb, axis=0, keepdims=True)

    @pl.when(r == 0)
    def _():
        gram_ref[...] = g
        cs_ref[...] = cs[None]

    @pl.when(r != 0)
    def _():
        gram_ref[...] += g
        cs_ref[...] += cs[None]

    @pl.when(r == _R1 - 1)
    def _():
        m_ref[...] = _make_m(gram_ref[...], cs_ref[0], perm_ref[0])[None]


def _moments_m(x, perms3):
    return pl.pallas_call(
        _fused_kernel,
        grid=(_S, _R1),
        in_specs=[
            pl.BlockSpec((_RB1, _D), lambda s, r: (s * _R1 + r, 0)),
            pl.BlockSpec((1, 1, _D), lambda s, r: (s, 0, 0)),
        ],
        out_specs=[
            pl.BlockSpec((1, _D, _D), lambda s, r: (s, 0, 0)),
            pl.BlockSpec((1, 1, _D), lambda s, r: (s, 0, 0)),
        ],
        out_shape=[
            jax.ShapeDtypeStruct((_S, _D, _D), jnp.bfloat16),
            jax.ShapeDtypeStruct((_S, 1, _D), jnp.float32),
        ],
        scratch_shapes=[pltpu.VMEM((_D, _D), jnp.float32)],
        compiler_params=pltpu.CompilerParams(
            dimension_semantics=("arbitrary", "arbitrary"),
            vmem_limit_bytes=56 * 1024 * 1024,
        ),
    )(x, perms3)


_RB3 = 2048
_T3 = (_S * _B) // _RB3
_VPB = _B // _RB3


def _apply_kernel(x_ref, m_ref, cs_ref, y_ref):
    xc = x_ref[...] - cs_ref[0] * (1.0 / _B)
    y_ref[...] = jax.lax.dot_general(
        xc.astype(jnp.bfloat16), m_ref[0],
        (((1,), (0,)), ((), ())),
        preferred_element_type=jnp.float32)


def _apply(x, m_bf, cs):
    return pl.pallas_call(
        _apply_kernel,
        grid=(_T3,),
        in_specs=[
            pl.BlockSpec((_RB3, _D), lambda t: (t, 0)),
            pl.BlockSpec((1, _D, _D), lambda t: (t // _VPB, 0, 0)),
            pl.BlockSpec((1, 1, _D), lambda t: (t // _VPB, 0, 0)),
        ],
        out_specs=pl.BlockSpec((_RB3, _D), lambda t: (t, 0)),
        out_shape=jax.ShapeDtypeStruct((_S * _B, _D), jnp.float32),
        compiler_params=pltpu.CompilerParams(
            dimension_semantics=("arbitrary",),
            vmem_limit_bytes=48 * 1024 * 1024,
        ),
    )(x, m_bf, cs)


def kernel(x, perms):
    m_bf, cs = _moments_m(x, perms.astype(jnp.int32)[:, None, :])
    return _apply(x, m_bf, cs)


# speedup vs baseline: 19.3327x; 1.0627x over previous
import jax
import jax.numpy as jnp
from jax.experimental import pallas as pl
from jax.experimental.pallas import tpu as pltpu

_S = 3
_B = 8192
_D = 1024
_G = 64
_d = 16
_NS_ITERS = 8

_RB1 = 2048
_R1 = _B // _RB1


def _make_m(gram, cs, perm_row):
    f32 = jnp.float32
    bf16 = jnp.bfloat16

    mu = cs * (1.0 / _B)
    mumu = jax.lax.dot_general(mu, mu, (((0,), (0,)), ((), ())),
                               preferred_element_type=f32,
                               precision=jax.lax.Precision.HIGHEST)
    covf = gram * (1.0 / _B) - mumu

    iota_r = jax.lax.broadcasted_iota(jnp.int32, (_D, _D), 0)
    iota_c = jax.lax.broadcasted_iota(jnp.int32, (_D, _D), 1)
    p_row = jnp.broadcast_to(perm_row, (_D, _D))
    e_bf = jnp.where(iota_r == p_row, f32(1), f32(0)).astype(bf16)

    t1 = jax.lax.dot_general(e_bf, covf.astype(bf16), (((0,), (0,)), ((), ())),
                             preferred_element_type=f32)
    cp = jax.lax.dot_general(t1.astype(bf16), e_bf, (((1,), (0,)), ((), ())),
                             preferred_element_type=f32)

    a2 = jnp.concatenate(
        [cp[g * _d:(g + 1) * _d, g * _d:(g + 1) * _d] for g in range(_G)],
        axis=1)

    blk_mask = (iota_r >> 4) == (iota_c >> 4)
    s_bf = jnp.where(blk_mask, f32(1), f32(0)).astype(bf16)
    rs = jnp.sum(a2 * a2, axis=0, keepdims=True)
    nrm = jax.lax.dot_general(rs.astype(bf16), s_bf, (((1,), (0,)), ((), ())),
                              preferred_element_type=f32) * 1.02
    inv_nrm = 1.0 / nrm

    eye2 = (jax.lax.broadcasted_iota(jnp.int32, (_d, _D), 0) ==
            (jax.lax.broadcasted_iota(jnp.int32, (_d, _D), 1) & (_d - 1))
            ).astype(f32)

    def _bd(q_bf):
        rep = pltpu.repeat(q_bf, _G, axis=0)
        return rep * s_bf

    pr = jax.lax.broadcasted_iota(jnp.int32, (128, 128), 0)
    pc = jax.lax.broadcasted_iota(jnp.int32, (128, 128), 1)
    s128_bf = jnp.where((pr >> 4) == (pc >> 4), f32(1), f32(0)).astype(bf16)

    def _bmm(p_bf, q_bf):
        outs = []
        for q in range(_D // 128):
            sl = slice(q * 128, (q + 1) * 128)
            bd = pltpu.repeat(q_bf[:, sl], 128 // _d, axis=0) * s128_bf
            outs.append(jax.lax.dot_general(
                p_bf[:, sl], bd, (((1,), (0,)), ((), ())),
                preferred_element_type=f32))
        return jnp.concatenate(outs, axis=1)

    y = a2 * inv_nrm
    z = eye2
    for _ in range(_NS_ITERS):
        y_bf = y.astype(bf16)
        t = 1.5 * eye2 - 0.5 * _bmm(z.astype(bf16), y_bf)
        t_bf = t.astype(bf16)
        y = _bmm(t_bf, y_bf)
        z = _bmm(t_bf, z.astype(bf16))
    w2 = z * jax.lax.rsqrt(nrm)

    t2 = jax.lax.dot_general(_bd(w2.astype(bf16)), e_bf,
                             (((1,), (1,)), ((), ())),
                             preferred_element_type=f32)
    m = jax.lax.dot_general(e_bf, t2.astype(bf16), (((1,), (0,)), ((), ())),
                            preferred_element_type=f32)
    return m.astype(bf16)


def _fused_kernel(x_ref, perm_ref, m_ref, cs_ref, gram_ref):
    r = pl.program_id(1)
    xb = x_ref[...]
    xh = xb.astype(jnp.bfloat16)
    g = jax.lax.dot_general(
        xh, xh, (((0,), (0,)), ((), ())),
        preferred_element_type=jnp.float32)
    cs = jnp.sum(xb, axis=0, keepdims=True)

    @pl.when(r == 0)
    def _():
        gram_ref[...] = g
        cs_ref[...] = cs[None]

    @pl.when(r != 0)
    def _():
        gram_ref[...] += g
        cs_ref[...] += cs[None]

    @pl.when(r == _R1 - 1)
    def _():
        m_ref[...] = _make_m(gram_ref[...], cs_ref[0], perm_ref[0])[None]


def _moments_m(x, perms3):
    return pl.pallas_call(
        _fused_kernel,
        grid=(_S, _R1),
        in_specs=[
            pl.BlockSpec((_RB1, _D), lambda s, r: (s * _R1 + r, 0)),
            pl.BlockSpec((1, 1, _D), lambda s, r: (s, 0, 0)),
        ],
        out_specs=[
            pl.BlockSpec((1, _D, _D), lambda s, r: (s, 0, 0)),
            pl.BlockSpec((1, 1, _D), lambda s, r: (s, 0, 0)),
        ],
        out_shape=[
            jax.ShapeDtypeStruct((_S, _D, _D), jnp.bfloat16),
            jax.ShapeDtypeStruct((_S, 1, _D), jnp.float32),
        ],
        scratch_shapes=[pltpu.VMEM((_D, _D), jnp.float32)],
        compiler_params=pltpu.CompilerParams(
            dimension_semantics=("arbitrary", "arbitrary"),
            vmem_limit_bytes=56 * 1024 * 1024,
        ),
    )(x, perms3)


_RB3 = 2048
_T3 = (_S * _B) // _RB3
_VPB = _B // _RB3


def _apply_kernel(x_ref, m_ref, cs_ref, y_ref):
    xc = x_ref[...] - cs_ref[0] * (1.0 / _B)
    y_ref[...] = jax.lax.dot_general(
        xc.astype(jnp.bfloat16), m_ref[0],
        (((1,), (0,)), ((), ())),
        preferred_element_type=jnp.float32)


def _apply(x, m_bf, cs):
    return pl.pallas_call(
        _apply_kernel,
        grid=(_T3,),
        in_specs=[
            pl.BlockSpec((_RB3, _D), lambda t: (t, 0)),
            pl.BlockSpec((1, _D, _D), lambda t: (t // _VPB, 0, 0)),
            pl.BlockSpec((1, 1, _D), lambda t: (t // _VPB, 0, 0)),
        ],
        out_specs=pl.BlockSpec((_RB3, _D), lambda t: (t, 0)),
        out_shape=jax.ShapeDtypeStruct((_S * _B, _D), jnp.float32),
        compiler_params=pltpu.CompilerParams(
            dimension_semantics=("arbitrary",),
            vmem_limit_bytes=48 * 1024 * 1024,
        ),
    )(x, m_bf, cs)


def kernel(x, perms):
    m_bf, cs = _moments_m(x, perms.astype(jnp.int32)[:, None, :])
    return _apply(x, m_bf, cs)


# speedup vs baseline: 20.4646x; 1.0585x over previous
import jax
import jax.numpy as jnp
from jax.experimental import pallas as pl
from jax.experimental.pallas import tpu as pltpu

_S = 3
_B = 8192
_D = 1024
_G = 64
_d = 16
_NS_ITERS = 8

_RB1 = 2048
_R1 = _B // _RB1


def _make_m(gram, cs, perm_row):
    f32 = jnp.float32
    bf16 = jnp.bfloat16

    mu = cs * (1.0 / _B)
    mumu = jax.lax.dot_general(mu, mu, (((0,), (0,)), ((), ())),
                               preferred_element_type=f32,
                               precision=jax.lax.Precision.HIGHEST)
    covf = gram * (1.0 / _B) - mumu

    iota_r = jax.lax.broadcasted_iota(jnp.int32, (_D, _D), 0)
    iota_c = jax.lax.broadcasted_iota(jnp.int32, (_D, _D), 1)
    p_row = jnp.broadcast_to(perm_row, (_D, _D))
    e_bf = jnp.where(iota_r == p_row, f32(1), f32(0)).astype(bf16)

    _NP = _D // 128
    _psl = [slice(q * 128, (q + 1) * 128) for q in range(_NP)]

    u = jax.lax.dot_general(covf.astype(bf16), e_bf, (((1,), (0,)), ((), ())),
                            preferred_element_type=f32)
    u_bf = u.astype(bf16)

    a2_parts = []
    for q in range(_NP):
        pq = jax.lax.dot_general(e_bf[:, _psl[q]], u_bf[:, _psl[q]],
                                 (((0,), (0,)), ((), ())),
                                 preferred_element_type=f32)
        a2_parts.append(jnp.concatenate(
            [pq[h * _d:(h + 1) * _d, h * _d:(h + 1) * _d]
             for h in range(128 // _d)], axis=1))
    a2 = jnp.concatenate(a2_parts, axis=1)

    pr = jax.lax.broadcasted_iota(jnp.int32, (128, 128), 0)
    pc = jax.lax.broadcasted_iota(jnp.int32, (128, 128), 1)
    s128_bf = jnp.where((pr >> 4) == (pc >> 4), f32(1), f32(0)).astype(bf16)

    rs = jnp.sum(a2 * a2, axis=0, keepdims=True)
    rs_bf = rs.astype(bf16)
    nrm = jnp.concatenate(
        [jax.lax.dot_general(rs_bf[:, s], s128_bf, (((1,), (0,)), ((), ())),
                             preferred_element_type=f32) for s in _psl],
        axis=1) * 1.02
    inv_nrm = 1.0 / nrm

    eye2 = (jax.lax.broadcasted_iota(jnp.int32, (_d, _D), 0) ==
            (jax.lax.broadcasted_iota(jnp.int32, (_d, _D), 1) & (_d - 1))
            ).astype(f32)

    def _bmm(p_bf, q_bf):
        outs = []
        for sl in _psl:
            bd = pltpu.repeat(q_bf[:, sl], 128 // _d, axis=0) * s128_bf
            outs.append(jax.lax.dot_general(
                p_bf[:, sl], bd, (((1,), (0,)), ((), ())),
                preferred_element_type=f32))
        return jnp.concatenate(outs, axis=1)

    y = a2 * inv_nrm
    z = eye2
    for _ in range(_NS_ITERS):
        y_bf = y.astype(bf16)
        t = 1.5 * eye2 - 0.5 * _bmm(z.astype(bf16), y_bf)
        t_bf = t.astype(bf16)
        y = _bmm(t_bf, y_bf)
        z = _bmm(t_bf, z.astype(bf16))
    w2 = z * jax.lax.rsqrt(nrm)

    w2_bf = w2.astype(bf16)
    ew = jnp.concatenate(
        [jax.lax.dot_general(
            e_bf[:, sl],
            pltpu.repeat(w2_bf[:, sl], 128 // _d, axis=0) * s128_bf,
            (((1,), (0,)), ((), ())), preferred_element_type=f32)
         for sl in _psl], axis=1)
    m = jax.lax.dot_general(ew.astype(bf16), e_bf, (((1,), (1,)), ((), ())),
                            preferred_element_type=f32)
    return m.astype(bf16)


def _fused_kernel(x_ref, perm_ref, m_ref, cs_ref, gram_ref):
    r = pl.program_id(1)
    xb = x_ref[...]
    xh = xb.astype(jnp.bfloat16)
    g = jax.lax.dot_general(
        xh, xh, (((0,), (0,)), ((), ())),
        preferred_element_type=jnp.float32)
    cs = jnp.sum(xb, axis=0, keepdims=True)

    @pl.when(r == 0)
    def _():
        gram_ref[...] = g
        cs_ref[...] = cs[None]

    @pl.when(r != 0)
    def _():
        gram_ref[...] += g
        cs_ref[...] += cs[None]

    @pl.when(r == _R1 - 1)
    def _():
        m_ref[...] = _make_m(gram_ref[...], cs_ref[0], perm_ref[0])[None]


def _moments_m(x, perms3):
    return pl.pallas_call(
        _fused_kernel,
        grid=(_S, _R1),
        in_specs=[
            pl.BlockSpec((_RB1, _D), lambda s, r: (s * _R1 + r, 0)),
            pl.BlockSpec((1, 1, _D), lambda s, r: (s, 0, 0)),
        ],
        out_specs=[
            pl.BlockSpec((1, _D, _D), lambda s, r: (s, 0, 0)),
            pl.BlockSpec((1, 1, _D), lambda s, r: (s, 0, 0)),
        ],
        out_shape=[
            jax.ShapeDtypeStruct((_S, _D, _D), jnp.bfloat16),
            jax.ShapeDtypeStruct((_S, 1, _D), jnp.float32),
        ],
        scratch_shapes=[pltpu.VMEM((_D, _D), jnp.float32)],
        compiler_params=pltpu.CompilerParams(
            dimension_semantics=("arbitrary", "arbitrary"),
            vmem_limit_bytes=56 * 1024 * 1024,
        ),
    )(x, perms3)


_RB3 = 2048
_T3 = (_S * _B) // _RB3
_VPB = _B // _RB3


def _apply_kernel(x_ref, m_ref, cs_ref, y_ref):
    xc = x_ref[...] - cs_ref[0] * (1.0 / _B)
    y_ref[...] = jax.lax.dot_general(
        xc.astype(jnp.bfloat16), m_ref[0],
        (((1,), (0,)), ((), ())),
        preferred_element_type=jnp.float32)


def _apply(x, m_bf, cs):
    return pl.pallas_call(
        _apply_kernel,
        grid=(_T3,),
        in_specs=[
            pl.BlockSpec((_RB3, _D), lambda t: (t, 0)),
            pl.BlockSpec((1, _D, _D), lambda t: (t // _VPB, 0, 0)),
            pl.BlockSpec((1, 1, _D), lambda t: (t // _VPB, 0, 0)),
        ],
        out_specs=pl.BlockSpec((_RB3, _D), lambda t: (t, 0)),
        out_shape=jax.ShapeDtypeStruct((_S * _B, _D), jnp.float32),
        compiler_params=pltpu.CompilerParams(
            dimension_semantics=("arbitrary",),
            vmem_limit_bytes=48 * 1024 * 1024,
        ),
    )(x, m_bf, cs)


def kernel(x, perms):
    m_bf, cs = _moments_m(x, perms.astype(jnp.int32)[:, None, :])
    return _apply(x, m_bf, cs)


# speedup vs baseline: 21.5213x; 1.0516x over previous
import jax
import jax.numpy as jnp
from jax.experimental import pallas as pl
from jax.experimental.pallas import tpu as pltpu

_S = 3
_B = 8192
_D = 1024
_G = 64
_d = 16
_NS_ITERS = 8

_RB1 = 2048
_R1 = _B // _RB1


def _make_m(gram, cs, perm_row):
    f32 = jnp.float32
    bf16 = jnp.bfloat16

    mu = cs * (1.0 / _B)
    mumu = jax.lax.dot_general(mu, mu, (((0,), (0,)), ((), ())),
                               preferred_element_type=f32,
                               precision=jax.lax.Precision.HIGHEST)
    covf = gram * (1.0 / _B) - mumu

    iota_r = jax.lax.broadcasted_iota(jnp.int32, (_D, _D), 0)
    iota_c = jax.lax.broadcasted_iota(jnp.int32, (_D, _D), 1)
    p_row = jnp.broadcast_to(perm_row, (_D, _D))
    e_bf = jnp.where(iota_r == p_row, f32(1), f32(0)).astype(bf16)

    _NP = _D // 128
    _psl = [slice(q * 128, (q + 1) * 128) for q in range(_NP)]

    u = jax.lax.dot_general(covf.astype(bf16), e_bf, (((1,), (0,)), ((), ())),
                            preferred_element_type=f32)
    u_bf = u.astype(bf16)

    a2_parts = []
    for q in range(_NP):
        pq = jax.lax.dot_general(e_bf[:, _psl[q]], u_bf[:, _psl[q]],
                                 (((0,), (0,)), ((), ())),
                                 preferred_element_type=f32)
        a2_parts.append(jnp.concatenate(
            [pq[h * _d:(h + 1) * _d, h * _d:(h + 1) * _d]
             for h in range(128 // _d)], axis=1))
    a2 = jnp.concatenate(a2_parts, axis=1)

    pr = jax.lax.broadcasted_iota(jnp.int32, (128, 128), 0)
    pc = jax.lax.broadcasted_iota(jnp.int32, (128, 128), 1)
    s128_bf = jnp.where((pr >> 4) == (pc >> 4), f32(1), f32(0)).astype(bf16)

    rs = jnp.sum(a2 * a2, axis=0, keepdims=True)
    rs_bf = rs.astype(bf16)
    nrm = jnp.concatenate(
        [jax.lax.dot_general(rs_bf[:, s], s128_bf, (((1,), (0,)), ((), ())),
                             preferred_element_type=f32) for s in _psl],
        axis=1) * 1.02
    inv_nrm = 1.0 / nrm

    eye2 = (jax.lax.broadcasted_iota(jnp.int32, (_d, _D), 0) ==
            (jax.lax.broadcasted_iota(jnp.int32, (_d, _D), 1) & (_d - 1))
            ).astype(f32)

    def _bmm(p_bf, q_bf):
        outs = []
        for sl in _psl:
            bd = pltpu.repeat(q_bf[:, sl], 128 // _d, axis=0) * s128_bf
            outs.append(jax.lax.dot_general(
                p_bf[:, sl], bd, (((1,), (0,)), ((), ())),
                preferred_element_type=f32))
        return jnp.concatenate(outs, axis=1)

    y = a2 * inv_nrm
    z = eye2
    for _ in range(_NS_ITERS):
        y_bf = y.astype(bf16)
        t = 1.5 * eye2 - 0.5 * _bmm(z.astype(bf16), y_bf)
        t_bf = t.astype(bf16)
        y = _bmm(t_bf, y_bf)
        z = _bmm(t_bf, z.astype(bf16))
    w2 = z * jax.lax.rsqrt(nrm)

    w2_bf = w2.astype(bf16)
    ew = jnp.concatenate(
        [jax.lax.dot_general(
            e_bf[:, sl],
            pltpu.repeat(w2_bf[:, sl], 128 // _d, axis=0) * s128_bf,
            (((1,), (0,)), ((), ())), preferred_element_type=f32)
         for sl in _psl], axis=1)
    m = jax.lax.dot_general(ew.astype(bf16), e_bf, (((1,), (1,)), ((), ())),
                            preferred_element_type=f32)
    return m.astype(bf16)


def _fused_kernel(x_ref, perm_ref, y_ref, gram_ref, cs_ref, m_ref):
    r = pl.program_id(1)

    @pl.when(r < _R1)
    def _():
        xb = x_ref[...]
        xh = xb.astype(jnp.bfloat16)
        g = jax.lax.dot_general(
            xh, xh, (((0,), (0,)), ((), ())),
            preferred_element_type=jnp.float32)
        cs = jnp.sum(xb, axis=0, keepdims=True)

        @pl.when(r == 0)
        def _():
            gram_ref[...] = g
            cs_ref[...] = cs

        @pl.when(r != 0)
        def _():
            gram_ref[...] += g
            cs_ref[...] += cs

        @pl.when(r == _R1 - 1)
        def _():
            m_ref[...] = _make_m(gram_ref[...], cs_ref[...], perm_ref[0])

    @pl.when(r >= _R1)
    def _():
        xc = x_ref[...] - cs_ref[...] * (1.0 / _B)
        y_ref[...] = jax.lax.dot_general(
            xc.astype(jnp.bfloat16), m_ref[...],
            (((1,), (0,)), ((), ())),
            preferred_element_type=jnp.float32)


def kernel(x, perms):
    perms3 = perms.astype(jnp.int32)[:, None, :]
    return pl.pallas_call(
        _fused_kernel,
        grid=(_S, 2 * _R1),
        in_specs=[
            pl.BlockSpec(
                (_RB1, _D),
                lambda s, r: (s * _R1 + jnp.where(r < _R1, r, r - _R1), 0)),
            pl.BlockSpec((1, 1, _D), lambda s, r: (s, 0, 0)),
        ],
        out_specs=pl.BlockSpec(
            (_RB1, _D),
            lambda s, r: (s * _R1 + jnp.where(r < _R1, 0, r - _R1), 0)),
        out_shape=jax.ShapeDtypeStruct((_S * _B, _D), jnp.float32),
        scratch_shapes=[
            pltpu.VMEM((_D, _D), jnp.float32),
            pltpu.VMEM((1, _D), jnp.float32),
            pltpu.VMEM((_D, _D), jnp.bfloat16),
        ],
        compiler_params=pltpu.CompilerParams(
            dimension_semantics=("arbitrary", "arbitrary"),
            vmem_limit_bytes=56 * 1024 * 1024,
        ),
    )(x, perms3)


# speedup vs baseline: 22.1909x; 1.0311x over previous
import jax
import jax.numpy as jnp
from jax.experimental import pallas as pl
from jax.experimental.pallas import tpu as pltpu

_S = 3
_B = 8192
_D = 1024
_G = 64
_d = 16
_NS_ITERS = 8

_RB1 = 2048
_R1 = _B // _RB1


def _make_m(gram, cs, perm_row):
    f32 = jnp.float32
    bf16 = jnp.bfloat16

    mu = cs * (1.0 / _B)
    mh_bf = mu.astype(bf16)
    ml_bf = (mu - mh_bf.astype(f32)).astype(bf16)
    _outer = lambda a, b: jax.lax.dot_general(
        a, b, (((0,), (0,)), ((), ())), preferred_element_type=f32)
    mumu = (_outer(mh_bf, mh_bf) + _outer(mh_bf, ml_bf)
            + _outer(ml_bf, mh_bf))
    covf = gram * (1.0 / _B) - mumu

    iota_r = jax.lax.broadcasted_iota(jnp.int32, (_D, _D), 0)
    iota_c = jax.lax.broadcasted_iota(jnp.int32, (_D, _D), 1)
    p_row = jnp.broadcast_to(perm_row, (_D, _D))
    e_bf = jnp.where(iota_r == p_row, f32(1), f32(0)).astype(bf16)

    _NP = _D // 128
    _psl = [slice(q * 128, (q + 1) * 128) for q in range(_NP)]

    u = jax.lax.dot_general(covf.astype(bf16), e_bf, (((1,), (0,)), ((), ())),
                            preferred_element_type=f32)
    u_bf = u.astype(bf16)

    a2_parts = []
    for q in range(_NP):
        pq = jax.lax.dot_general(e_bf[:, _psl[q]], u_bf[:, _psl[q]],
                                 (((0,), (0,)), ((), ())),
                                 preferred_element_type=f32)
        a2_parts.append(jnp.concatenate(
            [pq[h * _d:(h + 1) * _d, h * _d:(h + 1) * _d]
             for h in range(128 // _d)], axis=1))
    a2 = jnp.concatenate(a2_parts, axis=1)

    pr = jax.lax.broadcasted_iota(jnp.int32, (128, 128), 0)
    pc = jax.lax.broadcasted_iota(jnp.int32, (128, 128), 1)
    s128_bf = jnp.where((pr >> 4) == (pc >> 4), f32(1), f32(0)).astype(bf16)

    rs = jnp.sum(a2 * a2, axis=0, keepdims=True)
    rs_bf = rs.astype(bf16)
    nrm = jnp.concatenate(
        [jax.lax.dot_general(rs_bf[:, s], s128_bf, (((1,), (0,)), ((), ())),
                             preferred_element_type=f32) for s in _psl],
        axis=1) * 1.02
    inv_nrm = 1.0 / nrm

    eye2 = (jax.lax.broadcasted_iota(jnp.int32, (_d, _D), 0) ==
            (jax.lax.broadcasted_iota(jnp.int32, (_d, _D), 1) & (_d - 1))
            ).astype(f32)

    def _bmm(p_bf, q_bf):
        outs = []
        for sl in _psl:
            bd = pltpu.repeat(q_bf[:, sl], 128 // _d, axis=0) * s128_bf
            outs.append(jax.lax.dot_general(
                p_bf[:, sl], bd, (((1,), (0,)), ((), ())),
                preferred_element_type=f32))
        return jnp.concatenate(outs, axis=1)

    y = a2 * inv_nrm
    z = eye2
    for _ in range(_NS_ITERS):
        y_bf = y.astype(bf16)
        t = 1.5 * eye2 - 0.5 * _bmm(z.astype(bf16), y_bf)
        t_bf = t.astype(bf16)
        y = _bmm(t_bf, y_bf)
        z = _bmm(t_bf, z.astype(bf16))
    w2 = z * jax.lax.rsqrt(nrm)

    w2_bf = w2.astype(bf16)
    ew = jnp.concatenate(
        [jax.lax.dot_general(
            e_bf[:, sl],
            pltpu.repeat(w2_bf[:, sl], 128 // _d, axis=0) * s128_bf,
            (((1,), (0,)), ((), ())), preferred_element_type=f32)
         for sl in _psl], axis=1)
    m = jax.lax.dot_general(ew.astype(bf16), e_bf, (((1,), (1,)), ((), ())),
                            preferred_element_type=f32)
    return m.astype(bf16)


def _fused_kernel(x_ref, perm_ref, y_ref, gram_ref, cs_ref, m_ref):
    r = pl.program_id(1)

    @pl.when(r < _R1)
    def _():
        xb = x_ref[...]
        xh = xb.astype(jnp.bfloat16)
        g = jax.lax.dot_general(
            xh, xh, (((0,), (0,)), ((), ())),
            preferred_element_type=jnp.float32)
        cs = jnp.sum(xb, axis=0, keepdims=True)

        @pl.when(r == 0)
        def _():
            gram_ref[...] = g
            cs_ref[...] = cs

        @pl.when(r != 0)
        def _():
            gram_ref[...] += g
            cs_ref[...] += cs

        @pl.when(r == _R1 - 1)
        def _():
            m_ref[...] = _make_m(gram_ref[...], cs_ref[...], perm_ref[0])

    @pl.when(r >= _R1)
    def _():
        xc = x_ref[...] - cs_ref[...] * (1.0 / _B)
        y_ref[...] = jax.lax.dot_general(
            xc.astype(jnp.bfloat16), m_ref[...],
            (((1,), (0,)), ((), ())),
            preferred_element_type=jnp.float32)


def kernel(x, perms):
    perms3 = perms.astype(jnp.int32)[:, None, :]
    return pl.pallas_call(
        _fused_kernel,
        grid=(_S, 2 * _R1),
        in_specs=[
            pl.BlockSpec(
                (_RB1, _D),
                lambda s, r: (s * _R1 + jnp.where(r < _R1, r, r - _R1), 0)),
            pl.BlockSpec((1, 1, _D), lambda s, r: (s, 0, 0)),
        ],
        out_specs=pl.BlockSpec(
            (_RB1, _D),
            lambda s, r: (s * _R1 + jnp.where(r < _R1, 0, r - _R1), 0)),
        out_shape=jax.ShapeDtypeStruct((_S * _B, _D), jnp.float32),
        scratch_shapes=[
            pltpu.VMEM((_D, _D), jnp.float32),
            pltpu.VMEM((1, _D), jnp.float32),
            pltpu.VMEM((_D, _D), jnp.bfloat16),
        ],
        compiler_params=pltpu.CompilerParams(
            dimension_semantics=("arbitrary", "arbitrary"),
            vmem_limit_bytes=56 * 1024 * 1024,
        ),
    )(x, perms3)
